# Initial kernel scaffold; baseline (speedup 1.0000x reference)
import jax
import jax.numpy as jnp
from jax.experimental import pallas as pl


def kernel(x, table, W, b):
    raise NotImplementedError("write your pallas kernel here")


# speedup vs baseline: 34.5597x; 34.5597x over previous
import functools

import jax
import jax.numpy as jnp
from jax import lax
from jax.experimental import pallas as pl
from jax.experimental.pallas import tpu as pltpu
from jax.experimental.pallas import tpu_sc as plsc

BATCH = 4096
SEQ = 200
EMBED = 32
LANES = 16

NUM_CORES = 2
NUM_SUBCORES = 16
NW = NUM_CORES * NUM_SUBCORES
RPW = BATCH // NW
G = 8
NGRP = RPW // G
IDX_PER_G = G * SEQ
CHUNK = 128
NCH = (IDX_PER_G + CHUNK - 1) // CHUNK


def _body(table_hbm, x_hbm, w_hbm, b_hbm, out_hbm,
          idx_v, rows_v, w_v, b_v, out_v, sem_r):
    c = lax.axis_index("c")
    s = lax.axis_index("s")
    wid = s * NUM_CORES + c
    base = wid * RPW

    pltpu.sync_copy(w_hbm, w_v)
    pltpu.sync_copy(b_hbm, b_v)

    lane = jnp.arange(LANES, dtype=jnp.int32)
    zero = jnp.zeros((LANES,), jnp.float32)

    def pair(i, carry):
        y = zero
        for h in range(2):
            g = 2 * i + h
            goff = (base + g * G) * SEQ
            pltpu.sync_copy(x_hbm.at[pl.ds(goff, IDX_PER_G)], idx_v)
            cps = []
            for j in range(NCH):
                sz = min(CHUNK, IDX_PER_G - j * CHUNK)
                cps.append(pltpu.async_copy(
                    table_hbm.at[idx_v.at[pl.ds(j * CHUNK, sz)]],
                    rows_v.at[pl.ds(j * CHUNK, sz), :],
                    sem_r))
            for cp in cps:
                cp.wait()

            def sbody(si, accs):
                w0 = w_v[si, pl.ds(0, LANES)]
                w1 = w_v[si, pl.ds(LANES, LANES)]
                nxt = []
                for r in range(G):
                    a0 = accs[2 * r] + rows_v[r * SEQ + si, pl.ds(0, LANES)] * w0
                    a1 = accs[2 * r + 1] + rows_v[r * SEQ + si, pl.ds(LANES, LANES)] * w1
                    nxt += [a0, a1]
                return tuple(nxt)

            accs = lax.fori_loop(0, SEQ, sbody, (zero,) * (2 * G))
            for r in range(G):
                v = accs[2 * r] + accs[2 * r + 1]
                for d in (8, 4, 2, 1):
                    perm = jnp.bitwise_xor(lane, d)
                    v = v + v.at[perm].get(mode="promise_in_bounds")
                y = jnp.where(lane == (h * G + r), v, y)
        y = 1.0 / (1.0 + jnp.exp(-(y + b_v[...])))
        out_v[pl.ds(i * LANES, LANES)] = y
        return carry

    lax.fori_loop(0, RPW // LANES, pair, 0)
    pltpu.sync_copy(out_v, out_hbm.at[pl.ds(base, RPW)])


@jax.jit
def kernel(x, table, W, b):
    xf = x.reshape(-1).astype(jnp.int32)
    Wr = W.reshape(SEQ, EMBED).astype(jnp.float32)
    b16 = jnp.broadcast_to(b.astype(jnp.float32).reshape(()), (LANES,))
    mesh = plsc.VectorSubcoreMesh(core_axis_name="c", subcore_axis_name="s")
    k = pl.kernel(
        _body,
        out_type=jax.ShapeDtypeStruct((BATCH,), jnp.float32),
        mesh=mesh,
        compiler_params=pltpu.CompilerParams(use_tc_tiling_on_sc=False),
        scratch_types=[
            pltpu.VMEM((IDX_PER_G,), jnp.int32),
            pltpu.VMEM((IDX_PER_G, EMBED), jnp.float32),
            pltpu.VMEM((SEQ, EMBED), jnp.float32),
            pltpu.VMEM((LANES,), jnp.float32),
            pltpu.VMEM((RPW,), jnp.float32),
            pltpu.SemaphoreType.DMA,
        ],
    )
    out = k(table, xf, Wr, b16)
    return out.reshape(BATCH, 1)


# speedup vs baseline: 37.0049x; 1.0708x over previous
import functools

import jax
import jax.numpy as jnp
from jax import lax
from jax.experimental import pallas as pl
from jax.experimental.pallas import tpu as pltpu
from jax.experimental.pallas import tpu_sc as plsc

BATCH = 4096
SEQ = 200
EMBED = 32
LANES = 16

NUM_CORES = 2
NUM_SUBCORES = 16
NW = NUM_CORES * NUM_SUBCORES
RPW = BATCH // NW
G = 8
NGRP = RPW // G
NPAIR = NGRP // 2
IDX_PER_G = G * SEQ
CHUNK = 128
NCH = (IDX_PER_G + CHUNK - 1) // CHUNK


def _body(table_hbm, x_hbm, w_hbm, b_hbm, out_hbm,
          idx0_v, idx1_v, rows0_v, rows1_v, w_v, b_v, out_v,
          sem_i0, sem_i1, sem_r0, sem_r1):
    c = lax.axis_index("c")
    s = lax.axis_index("s")
    wid = s * NUM_CORES + c
    base = wid * RPW

    idx_bufs = (idx0_v, idx1_v)
    rows_bufs = (rows0_v, rows1_v)
    sem_i = (sem_i0, sem_i1)
    sem_r = (sem_r0, sem_r1)

    pltpu.sync_copy(w_hbm, w_v)
    pltpu.sync_copy(b_hbm, b_v)

    lane = jnp.arange(LANES, dtype=jnp.int32)
    zero = jnp.zeros((LANES,), jnp.float32)

    def start_idx(g, h):
        pltpu.make_async_copy(
            x_hbm.at[pl.ds((base + g * G) * SEQ, IDX_PER_G)],
            idx_bufs[h], sem_i[h]).start()

    def wait_idx(h):
        pltpu.make_async_copy(
            x_hbm.at[pl.ds(base * SEQ, IDX_PER_G)],
            idx_bufs[h], sem_i[h]).wait()

    def fire_gathers(h):
        for j in range(NCH):
            sz = min(CHUNK, IDX_PER_G - j * CHUNK)
            pltpu.make_async_copy(
                table_hbm.at[idx_bufs[h].at[pl.ds(j * CHUNK, sz)]],
                rows_bufs[h].at[pl.ds(j * CHUNK, sz), :],
                sem_r[h]).start()

    def wait_gathers(h):
        for j in range(NCH):
            sz = min(CHUNK, IDX_PER_G - j * CHUNK)
            pltpu.make_async_copy(
                table_hbm.at[idx_bufs[h].at[pl.ds(j * CHUNK, sz)]],
                rows_bufs[h].at[pl.ds(j * CHUNK, sz), :],
                sem_r[h]).wait()

    def compute_group(h, lane_off):
        rows_v = rows_bufs[h]

        def sbody(si, accs):
            w0 = w_v[si, pl.ds(0, LANES)]
            w1 = w_v[si, pl.ds(LANES, LANES)]
            nxt = []
            for r in range(G):
                a0 = accs[2 * r] + rows_v[r * SEQ + si, pl.ds(0, LANES)] * w0
                a1 = accs[2 * r + 1] + rows_v[r * SEQ + si, pl.ds(LANES, LANES)] * w1
                nxt += [a0, a1]
            return tuple(nxt)

        accs = lax.fori_loop(0, SEQ, sbody, (zero,) * (2 * G), unroll=2)
        y = zero
        for r in range(G):
            v = accs[2 * r] + accs[2 * r + 1]
            for d in (8, 4, 2, 1):
                perm = jnp.bitwise_xor(lane, d)
                v = v + v.at[perm].get(mode="promise_in_bounds")
            y = jnp.where(lane == (lane_off + r), v, y)
        return y

    start_idx(0, 0)
    wait_idx(0)
    fire_gathers(0)
    start_idx(1, 1)

    def pair(i, carry):
        g0 = 2 * i
        wait_gathers(0)
        wait_idx(1)
        fire_gathers(1)

        @pl.when(i < NPAIR - 1)
        def _():
            start_idx(g0 + 2, 0)

        ylo = compute_group(0, 0)

        wait_gathers(1)

        @pl.when(i < NPAIR - 1)
        def _():
            wait_idx(0)
            fire_gathers(0)
            start_idx(g0 + 3, 1)

        yhi = compute_group(1, G)

        y = ylo + yhi
        y = 1.0 / (1.0 + jnp.exp(-(y + b_v[...])))
        out_v[pl.ds(i * LANES, LANES)] = y
        return carry

    lax.fori_loop(0, NPAIR, pair, 0)
    pltpu.sync_copy(out_v, out_hbm.at[pl.ds(base, RPW)])


@jax.jit
def kernel(x, table, W, b):
    xf = x.reshape(-1).astype(jnp.int32)
    Wr = W.reshape(SEQ, EMBED).astype(jnp.float32)
    b16 = jnp.broadcast_to(b.astype(jnp.float32).reshape(()), (LANES,))
    mesh = plsc.VectorSubcoreMesh(core_axis_name="c", subcore_axis_name="s")
    k = pl.kernel(
        _body,
        out_type=jax.ShapeDtypeStruct((BATCH,), jnp.float32),
        mesh=mesh,
        compiler_params=pltpu.CompilerParams(use_tc_tiling_on_sc=False),
        scratch_types=[
            pltpu.VMEM((IDX_PER_G,), jnp.int32),
            pltpu.VMEM((IDX_PER_G,), jnp.int32),
            pltpu.VMEM((IDX_PER_G, EMBED), jnp.float32),
            pltpu.VMEM((IDX_PER_G, EMBED), jnp.float32),
            pltpu.VMEM((SEQ, EMBED), jnp.float32),
            pltpu.VMEM((LANES,), jnp.float32),
            pltpu.VMEM((RPW,), jnp.float32),
            pltpu.SemaphoreType.DMA,
            pltpu.SemaphoreType.DMA,
            pltpu.SemaphoreType.DMA,
            pltpu.SemaphoreType.DMA,
        ],
    )
    out = k(table, xf, Wr, b16)
    return out.reshape(BATCH, 1)
